# Initial kernel scaffold; baseline (speedup 1.0000x reference)
import jax
import jax.numpy as jnp
from jax.experimental import pallas as pl


def kernel(x, edge_index, W1, b1, W2, b2, gamma, beta, lin1_W, lin1_b, lin2_W, lin2_b):
    raise NotImplementedError("write your pallas kernel here")


# speedup vs baseline: 96.2058x; 96.2058x over previous
import functools

import jax
import jax.numpy as jnp
from jax import lax
from jax.experimental import pallas as pl
from jax.experimental.pallas import tpu as pltpu
from jax.experimental.pallas import tpu_sc as plsc

N = 10000
E = 160000
NPAD = 10240
NS = 16
EPT = E // NS
SLICE = NPAD // NS
H1 = 2048
H2 = 1024
H3 = 256
OUT = 124
EPS = 1e-5
NEG_SLOPE = 0.01


def _sc_body(src_h, dst_h, x_h, s2_h, t_h,
             src_v, dst_v, tab_x, tab_d, tab_s1, norm_v,
             acc_a, acc_b, colbuf, tmp_v, sh_acc, sh_full):
    tid = lax.axis_index("s")
    base_e = tid * EPT
    base_n = tid * SLICE
    zeros16 = jnp.zeros((16,), jnp.float32)
    ones16 = jnp.ones((16,), jnp.float32)

    pltpu.sync_copy(src_h.at[pl.ds(base_e, EPT)], src_v)
    pltpu.sync_copy(dst_h.at[pl.ds(base_e, EPT)], dst_v)
    pltpu.sync_copy(x_h, tab_x.at[pl.ds(0, N)])

    def zero(ref):
        @pl.loop(0, NPAD // 16)
        def _(i):
            ref[pl.ds(i * 16, 16)] = zeros16

    def reduce_acc(acc_ref, out_ref):
        pltpu.sync_copy(acc_ref, sh_acc.at[tid])
        plsc.subcore_barrier()
        for j in range(NS):
            pltpu.sync_copy(sh_acc.at[j, pl.ds(base_n, SLICE)], colbuf.at[j])

        @pl.loop(0, SLICE // 16)
        def _(k):
            acc = colbuf[0, pl.ds(k * 16, 16)]
            for j in range(1, NS):
                acc = acc + colbuf[j, pl.ds(k * 16, 16)]
            out_ref[pl.ds(k * 16, 16)] = acc

        plsc.subcore_barrier()

    zero(acc_a)
    zero(acc_b)

    @pl.loop(0, EPT // 16)
    def _(i):
        d16 = dst_v[pl.ds(i * 16, 16)]
        plsc.addupdate_scatter(acc_a, [d16], ones16)

    reduce_acc(acc_a, tmp_v)

    @pl.loop(0, SLICE // 16)
    def _(k):
        deg = tmp_v[pl.ds(k * 16, 16)] + 1.0
        i32 = plsc.bitcast(deg, jnp.int32)
        i32 = jnp.int32(0x5F3759DF) - lax.shift_right_logical(i32, 1)
        y = plsc.bitcast(i32, jnp.float32)
        half = deg * 0.5
        for _ in range(3):
            y = y * (1.5 - half * y * y)
        tmp_v[pl.ds(k * 16, 16)] = y

    pltpu.sync_copy(tmp_v, sh_full.at[pl.ds(base_n, SLICE)])
    plsc.subcore_barrier()
    pltpu.sync_copy(sh_full, tab_d)
    plsc.subcore_barrier()

    zero(acc_a)
    zero(acc_b)

    @pl.loop(0, EPT // 16)
    def _(i):
        s16 = src_v[pl.ds(i * 16, 16)]
        d16 = dst_v[pl.ds(i * 16, 16)]
        dv_s = plsc.load_gather(tab_d, [s16])
        dv_d = plsc.load_gather(tab_d, [d16])
        nrm = dv_s * dv_d
        norm_v[pl.ds(i * 16, 16)] = nrm
        xv = plsc.load_gather(tab_x, [s16])
        plsc.addupdate_scatter(acc_a, [d16], nrm * xv)
        plsc.addupdate_scatter(acc_b, [d16], nrm)

    reduce_acc(acc_a, tmp_v)

    @pl.loop(0, SLICE // 16)
    def _(k):
        dv = tab_d[pl.ds(base_n + k * 16, 16)]
        xv = tab_x[pl.ds(base_n + k * 16, 16)]
        tmp_v[pl.ds(k * 16, 16)] = tmp_v[pl.ds(k * 16, 16)] + dv * dv * xv

    pltpu.sync_copy(tmp_v, sh_full.at[pl.ds(base_n, SLICE)])
    plsc.subcore_barrier()
    pltpu.sync_copy(sh_full, tab_s1)
    plsc.subcore_barrier()

    reduce_acc(acc_b, tmp_v)

    @pl.loop(0, SLICE // 16)
    def _(k):
        dv = tab_d[pl.ds(base_n + k * 16, 16)]
        tmp_v[pl.ds(k * 16, 16)] = tmp_v[pl.ds(k * 16, 16)] + dv * dv

    pltpu.sync_copy(tmp_v, t_h.at[pl.ds(base_n, SLICE)])

    zero(acc_a)

    @pl.loop(0, EPT // 16)
    def _(i):
        s16 = src_v[pl.ds(i * 16, 16)]
        d16 = dst_v[pl.ds(i * 16, 16)]
        nrm = norm_v[pl.ds(i * 16, 16)]
        s1v = plsc.load_gather(tab_s1, [s16])
        plsc.addupdate_scatter(acc_a, [d16], nrm * s1v)

    reduce_acc(acc_a, tmp_v)

    @pl.loop(0, SLICE // 16)
    def _(k):
        dv = tab_d[pl.ds(base_n + k * 16, 16)]
        s1v = tab_s1[pl.ds(base_n + k * 16, 16)]
        tmp_v[pl.ds(k * 16, 16)] = tmp_v[pl.ds(k * 16, 16)] + dv * dv * s1v

    pltpu.sync_copy(tmp_v, s2_h.at[pl.ds(base_n, SLICE)])


_sc_graph = pl.kernel(
    _sc_body,
    out_type=(
        jax.ShapeDtypeStruct((NPAD,), jnp.float32),
        jax.ShapeDtypeStruct((NPAD,), jnp.float32),
    ),
    mesh=plsc.VectorSubcoreMesh(
        core_axis_name="c", subcore_axis_name="s", num_cores=1),
    compiler_params=pltpu.CompilerParams(needs_layout_passes=False),
    scratch_types=[
        pltpu.VMEM((EPT,), jnp.int32),
        pltpu.VMEM((EPT,), jnp.int32),
        pltpu.VMEM((NPAD,), jnp.float32),
        pltpu.VMEM((NPAD,), jnp.float32),
        pltpu.VMEM((NPAD,), jnp.float32),
        pltpu.VMEM((EPT,), jnp.float32),
        pltpu.VMEM((NPAD,), jnp.float32),
        pltpu.VMEM((NPAD,), jnp.float32),
        pltpu.VMEM((NS, SLICE), jnp.float32),
        pltpu.VMEM((SLICE,), jnp.float32),
        pltpu.VMEM_SHARED((NS, NPAD), jnp.float32),
        pltpu.VMEM_SHARED((NPAD,), jnp.float32),
    ],
)


def _prep_body(s2_ref, t_ref, w1_ref, b1_ref, W2_ref, gamma_ref, beta_ref,
               l1w_ref, l1b_ref, p_ref, q_ref, r_ref):
    rows = lax.broadcasted_iota(jnp.int32, (NPAD // 128, 128), 0)
    cols = lax.broadcasted_iota(jnp.int32, (NPAD // 128, 128), 1)
    mask = (rows * 128 + cols) < N

    s2 = jnp.where(mask, s2_ref[...], 0.0)
    t = jnp.where(mask, t_ref[...], 0.0)
    inv_n = 1.0 / N
    m_s = jnp.sum(s2) * inv_n
    m_t = jnp.sum(t) * inv_n
    ds = jnp.where(mask, s2 - m_s, 0.0)
    dt = jnp.where(mask, t - m_t, 0.0)
    vs = jnp.sum(ds * ds) * inv_n
    vt = jnp.sum(dt * dt) * inv_n
    cv = jnp.sum(ds * dt) * inv_n

    u = jnp.dot(w1_ref[...], W2_ref[...], preferred_element_type=jnp.float32)
    c = jnp.dot(b1_ref[...], W2_ref[...], preferred_element_type=jnp.float32)
    var = vs * u * u + vt * c * c + 2.0 * cv * u * c
    scale = gamma_ref[...] / jnp.sqrt(var + EPS)

    p = jnp.dot(u * scale, l1w_ref[...], preferred_element_type=jnp.float32)
    q = jnp.dot(c * scale, l1w_ref[...], preferred_element_type=jnp.float32)
    r = jnp.dot(beta_ref[...], l1w_ref[...],
                preferred_element_type=jnp.float32) + l1b_ref[...]
    p_ref[...] = p
    q_ref[...] = q
    r_ref[...] = r - m_s * p - m_t * q


_prep = pl.pallas_call(
    _prep_body,
    out_shape=(
        jax.ShapeDtypeStruct((1, H3), jnp.float32),
        jax.ShapeDtypeStruct((1, H3), jnp.float32),
        jax.ShapeDtypeStruct((1, H3), jnp.float32),
    ),
)


ROWS_BLK = 1000


def _main_body(s2_ref, t_ref, p_ref, q_ref, r_ref, l2w_ref, l2b_ref, o_ref):
    h = s2_ref[...] * p_ref[...] + t_ref[...] * q_ref[...] + r_ref[...]
    h = jnp.where(h > 0, h, NEG_SLOPE * h)
    logits = jnp.dot(h, l2w_ref[...],
                     preferred_element_type=jnp.float32) + l2b_ref[...]
    m = jnp.max(logits, axis=1, keepdims=True)
    z = logits - m
    lse = jnp.log(jnp.sum(jnp.exp(z), axis=1, keepdims=True))
    o_ref[...] = z - lse


_main = pl.pallas_call(
    _main_body,
    grid=(N // ROWS_BLK,),
    in_specs=[
        pl.BlockSpec((ROWS_BLK, 1), lambda i: (i, 0)),
        pl.BlockSpec((ROWS_BLK, 1), lambda i: (i, 0)),
        pl.BlockSpec((1, H3), lambda i: (0, 0)),
        pl.BlockSpec((1, H3), lambda i: (0, 0)),
        pl.BlockSpec((1, H3), lambda i: (0, 0)),
        pl.BlockSpec((H3, OUT), lambda i: (0, 0)),
        pl.BlockSpec((1, OUT), lambda i: (0, 0)),
    ],
    out_specs=pl.BlockSpec((ROWS_BLK, OUT), lambda i: (i, 0)),
    out_shape=jax.ShapeDtypeStruct((N, OUT), jnp.float32),
)


def kernel(x, edge_index, W1, b1, W2, b2, gamma, beta, lin1_W, lin1_b,
           lin2_W, lin2_b):
    del b2
    src = edge_index[0]
    dst = edge_index[1]
    xf = x[:, 0].astype(jnp.float32)

    s2p, tp = _sc_graph(src, dst, xf)

    p, q, r = _prep(
        s2p.reshape(NPAD // 128, 128),
        tp.reshape(NPAD // 128, 128),
        W1.reshape(1, H1),
        b1.reshape(1, H1),
        W2,
        gamma.reshape(1, H2),
        beta.reshape(1, H2),
        lin1_W,
        lin1_b.reshape(1, H3),
    )

    return _main(
        s2p[:N].reshape(N, 1),
        tp[:N].reshape(N, 1),
        p, q, r,
        lin2_W,
        lin2_b.reshape(1, OUT),
    )


# speedup vs baseline: 116.7465x; 1.2135x over previous
import jax
import jax.numpy as jnp
from jax import lax
from jax.experimental import pallas as pl
from jax.experimental.pallas import tpu as pltpu
from jax.experimental.pallas import tpu_sc as plsc

N = 10000
E = 160000
NPAD = 10240
NS = 16
EPT = E // NS
SLICE = NPAD // NS
H1 = 2048
H2 = 1024
H3 = 256
OUT = 124
EPS = 1e-5
NEG_SLOPE = 0.01


def _sc_body(src_h, dst_h, x_h, s2_h, t_h, hacc, hb_dinv, hb_s1,
             src_v, dst_v, tab_x, tab_d, tab_s1, norm_v,
             acc_a, acc_b, colbuf, tmp_v):
    tid = lax.axis_index("s")
    base_e = tid * EPT
    base_n = tid * SLICE
    zeros16 = jnp.zeros((16,), jnp.float32)
    ones16 = jnp.ones((16,), jnp.float32)

    pltpu.sync_copy(src_h.at[pl.ds(base_e, EPT)], src_v)
    pltpu.sync_copy(dst_h.at[pl.ds(base_e, EPT)], dst_v)
    pltpu.sync_copy(x_h, tab_x.at[pl.ds(0, N)])

    def zero(ref):
        @pl.loop(0, NPAD // 16, unroll=4)
        def _(i):
            ref[pl.ds(i * 16, 16)] = zeros16

    def reduce_acc(acc_ref, out_ref):
        pltpu.sync_copy(acc_ref, hacc.at[tid])
        plsc.subcore_barrier()
        pltpu.sync_copy(hacc.at[:, pl.ds(base_n, SLICE)], colbuf)

        @pl.loop(0, SLICE // 16, unroll=2)
        def _(k):
            acc = colbuf[0, pl.ds(k * 16, 16)]
            for j in range(1, NS):
                acc = acc + colbuf[j, pl.ds(k * 16, 16)]
            out_ref[pl.ds(k * 16, 16)] = acc

        plsc.subcore_barrier()

    zero(acc_a)
    zero(acc_b)

    @pl.loop(0, EPT // 16, unroll=4)
    def _(i):
        d16 = dst_v[pl.ds(i * 16, 16)]
        plsc.addupdate_scatter(acc_a, [d16], ones16)

    reduce_acc(acc_a, tmp_v)
    zero(acc_a)

    @pl.loop(0, SLICE // 16, unroll=2)
    def _(k):
        deg = tmp_v[pl.ds(k * 16, 16)] + 1.0
        i32 = plsc.bitcast(deg, jnp.int32)
        i32 = jnp.int32(0x5F3759DF) - lax.shift_right_logical(i32, 1)
        y = plsc.bitcast(i32, jnp.float32)
        half = deg * 0.5
        for _ in range(3):
            y = y * (1.5 - half * y * y)
        tmp_v[pl.ds(k * 16, 16)] = y

    pltpu.sync_copy(tmp_v, hb_dinv.at[pl.ds(base_n, SLICE)])
    plsc.subcore_barrier()
    pltpu.sync_copy(hb_dinv, tab_d)
    plsc.subcore_barrier()

    @pl.loop(0, EPT // 16, unroll=4)
    def _(i):
        s16 = src_v[pl.ds(i * 16, 16)]
        d16 = dst_v[pl.ds(i * 16, 16)]
        dv_s = plsc.load_gather(tab_d, [s16])
        dv_d = plsc.load_gather(tab_d, [d16])
        nrm = dv_s * dv_d
        norm_v[pl.ds(i * 16, 16)] = nrm
        xv = plsc.load_gather(tab_x, [s16])
        plsc.addupdate_scatter(acc_b, [d16], nrm * xv)
        plsc.addupdate_scatter(acc_a, [d16], nrm)

    reduce_acc(acc_b, tmp_v)
    zero(acc_b)

    @pl.loop(0, SLICE // 16, unroll=2)
    def _(k):
        dv = tab_d[pl.ds(base_n + k * 16, 16)]
        xv = tab_x[pl.ds(base_n + k * 16, 16)]
        tmp_v[pl.ds(k * 16, 16)] = tmp_v[pl.ds(k * 16, 16)] + dv * dv * xv

    pltpu.sync_copy(tmp_v, hb_s1.at[pl.ds(base_n, SLICE)])
    plsc.subcore_barrier()
    pltpu.sync_copy(hb_s1, tab_s1)
    plsc.subcore_barrier()

    reduce_acc(acc_a, tmp_v)

    @pl.loop(0, SLICE // 16, unroll=2)
    def _(k):
        dv = tab_d[pl.ds(base_n + k * 16, 16)]
        tmp_v[pl.ds(k * 16, 16)] = tmp_v[pl.ds(k * 16, 16)] + dv * dv

    pltpu.sync_copy(tmp_v, t_h.at[pl.ds(base_n, SLICE)])

    @pl.loop(0, EPT // 16, unroll=4)
    def _(i):
        s16 = src_v[pl.ds(i * 16, 16)]
        d16 = dst_v[pl.ds(i * 16, 16)]
        nrm = norm_v[pl.ds(i * 16, 16)]
        s1v = plsc.load_gather(tab_s1, [s16])
        plsc.addupdate_scatter(acc_b, [d16], nrm * s1v)

    reduce_acc(acc_b, tmp_v)

    @pl.loop(0, SLICE // 16, unroll=2)
    def _(k):
        dv = tab_d[pl.ds(base_n + k * 16, 16)]
        s1v = tab_s1[pl.ds(base_n + k * 16, 16)]
        tmp_v[pl.ds(k * 16, 16)] = tmp_v[pl.ds(k * 16, 16)] + dv * dv * s1v

    pltpu.sync_copy(tmp_v, s2_h.at[pl.ds(base_n, SLICE)])


_sc_graph = pl.kernel(
    _sc_body,
    out_type=(
        jax.ShapeDtypeStruct((NPAD,), jnp.float32),
        jax.ShapeDtypeStruct((NPAD,), jnp.float32),
        jax.ShapeDtypeStruct((NS, NPAD), jnp.float32),
        jax.ShapeDtypeStruct((NPAD,), jnp.float32),
        jax.ShapeDtypeStruct((NPAD,), jnp.float32),
    ),
    mesh=plsc.VectorSubcoreMesh(
        core_axis_name="c", subcore_axis_name="s", num_cores=1),
    compiler_params=pltpu.CompilerParams(needs_layout_passes=False),
    scratch_types=[
        pltpu.VMEM((EPT,), jnp.int32),
        pltpu.VMEM((EPT,), jnp.int32),
        pltpu.VMEM((NPAD,), jnp.float32),
        pltpu.VMEM((NPAD,), jnp.float32),
        pltpu.VMEM((NPAD,), jnp.float32),
        pltpu.VMEM((EPT,), jnp.float32),
        pltpu.VMEM((NPAD,), jnp.float32),
        pltpu.VMEM((NPAD,), jnp.float32),
        pltpu.VMEM((NS, SLICE), jnp.float32),
        pltpu.VMEM((SLICE,), jnp.float32),
    ],
)


def _uc_body(wb_ref, W2_ref, uc_ref):
    uc_ref[...] = jnp.dot(wb_ref[...], W2_ref[...],
                          preferred_element_type=jnp.float32)


_uc = pl.pallas_call(
    _uc_body,
    out_shape=jax.ShapeDtypeStruct((2, H2), jnp.float32),
)


ROWS_BLK = 1000


def _main_body(s2f_ref, tf_ref, uc_ref, gamma_ref, beta_ref, l1w_ref,
               l1b_ref, l2w_ref, l2b_ref, s2_ref, t_ref, o_ref,
               p_s, q_s, r_s):
    i = pl.program_id(0)

    @pl.when(i == 0)
    def _():
        rows = lax.broadcasted_iota(jnp.int32, (NPAD // 128, 128), 0)
        cols = lax.broadcasted_iota(jnp.int32, (NPAD // 128, 128), 1)
        mask = (rows * 128 + cols) < N

        s2 = jnp.where(mask, s2f_ref[...], 0.0)
        t = jnp.where(mask, tf_ref[...], 0.0)
        inv_n = 1.0 / N
        m_s = jnp.sum(s2) * inv_n
        m_t = jnp.sum(t) * inv_n
        ds = jnp.where(mask, s2 - m_s, 0.0)
        dt = jnp.where(mask, t - m_t, 0.0)
        vs = jnp.sum(ds * ds) * inv_n
        vt = jnp.sum(dt * dt) * inv_n
        cv = jnp.sum(ds * dt) * inv_n

        u = uc_ref[0:1, :]
        c = uc_ref[1:2, :]
        var = vs * u * u + vt * c * c + 2.0 * cv * u * c
        scale = gamma_ref[...] / jnp.sqrt(var + EPS)

        p = jnp.dot(u * scale, l1w_ref[...],
                    preferred_element_type=jnp.float32)
        q = jnp.dot(c * scale, l1w_ref[...],
                    preferred_element_type=jnp.float32)
        r = jnp.dot(beta_ref[...], l1w_ref[...],
                    preferred_element_type=jnp.float32) + l1b_ref[...]
        p_s[...] = p
        q_s[...] = q
        r_s[...] = r - m_s * p - m_t * q

    h = s2_ref[...] * p_s[...] + t_ref[...] * q_s[...] + r_s[...]
    h = jnp.where(h > 0, h, NEG_SLOPE * h)
    logits = jnp.dot(h, l2w_ref[...],
                     preferred_element_type=jnp.float32) + l2b_ref[...]
    m = jnp.max(logits, axis=1, keepdims=True)
    z = logits - m
    lse = jnp.log(jnp.sum(jnp.exp(z), axis=1, keepdims=True))
    o_ref[...] = z - lse


_main = pl.pallas_call(
    _main_body,
    grid=(N // ROWS_BLK,),
    in_specs=[
        pl.BlockSpec((NPAD // 128, 128), lambda i: (0, 0)),
        pl.BlockSpec((NPAD // 128, 128), lambda i: (0, 0)),
        pl.BlockSpec((2, H2), lambda i: (0, 0)),
        pl.BlockSpec((1, H2), lambda i: (0, 0)),
        pl.BlockSpec((1, H2), lambda i: (0, 0)),
        pl.BlockSpec((H2, H3), lambda i: (0, 0)),
        pl.BlockSpec((1, H3), lambda i: (0, 0)),
        pl.BlockSpec((H3, OUT), lambda i: (0, 0)),
        pl.BlockSpec((1, OUT), lambda i: (0, 0)),
        pl.BlockSpec((ROWS_BLK, 1), lambda i: (i, 0)),
        pl.BlockSpec((ROWS_BLK, 1), lambda i: (i, 0)),
    ],
    out_specs=pl.BlockSpec((ROWS_BLK, OUT), lambda i: (i, 0)),
    out_shape=jax.ShapeDtypeStruct((N, OUT), jnp.float32),
    scratch_shapes=[
        pltpu.VMEM((1, H3), jnp.float32),
        pltpu.VMEM((1, H3), jnp.float32),
        pltpu.VMEM((1, H3), jnp.float32),
    ],
)


def kernel(x, edge_index, W1, b1, W2, b2, gamma, beta, lin1_W, lin1_b,
           lin2_W, lin2_b):
    del b2
    src = edge_index[0]
    dst = edge_index[1]
    xf = x[:, 0].astype(jnp.float32)

    uc = _uc(jnp.concatenate([W1.reshape(1, H1), b1.reshape(1, H1)], axis=0),
             W2)
    s2p, tp, _, _, _ = _sc_graph(src, dst, xf)

    return _main(
        s2p.reshape(NPAD // 128, 128),
        tp.reshape(NPAD // 128, 128),
        uc,
        gamma.reshape(1, H2),
        beta.reshape(1, H2),
        lin1_W,
        lin1_b.reshape(1, H3),
        lin2_W,
        lin2_b.reshape(1, OUT),
        s2p.reshape(NPAD, 1),
        tp.reshape(NPAD, 1),
    )


# speedup vs baseline: 131.5003x; 1.1264x over previous
import jax
import jax.numpy as jnp
from jax import lax
from jax.experimental import pallas as pl
from jax.experimental.pallas import tpu as pltpu
from jax.experimental.pallas import tpu_sc as plsc

N = 10000
E = 160000
NS = 16
SLICE = 1024
NPAD = NS * SLICE
EPT = E // NS
SROWS = SLICE // 128
FULL_T = N // SLICE
REM = N - FULL_T * SLICE
OFF = SLICE - REM
H1 = 2048
H2 = 1024
H3 = 256
OUT = 124
EPS = 1e-5
NEG_SLOPE = 0.01


def _sc_body(ei_h, x_h, s2_h, t_h, hacc0, hacc2, hb2, hbz,
             src_v, dst_v, tab, tab_z, x_sv,
             acc, colbuf, tdinv, twx, tz, tmp_v, tmp2):
    tid = lax.axis_index("s")
    base_e = tid * EPT
    base_n = tid * SLICE
    zeros16 = jnp.zeros((16,), jnp.float32)
    ones16 = jnp.ones((16,), jnp.float32)

    pltpu.sync_copy(ei_h.at[pl.ds(base_e, EPT)], src_v)
    pltpu.sync_copy(ei_h.at[pl.ds(E + base_e, EPT)], dst_v)

    @pl.when(tid < FULL_T)
    def _():
        pltpu.sync_copy(x_h.at[pl.ds(base_n, SLICE)], x_sv)

    @pl.when(tid == FULL_T)
    def _():
        pltpu.sync_copy(x_h.at[pl.ds(N - SLICE, SLICE)], tmp_v)
        for k in range(REM // 16):
            x_sv[pl.ds(k * 16, 16)] = tmp_v[pl.ds(OFF + k * 16, 16)]
        for k in range(REM // 16, SLICE // 16):
            x_sv[pl.ds(k * 16, 16)] = zeros16

    @pl.when(tid > FULL_T)
    def _():
        for k in range(SLICE // 16):
            x_sv[pl.ds(k * 16, 16)] = zeros16

    def zero(ref, n):
        @pl.loop(0, n // 16, unroll=4)
        def _(i):
            ref[pl.ds(i * 16, 16)] = zeros16

    def reduce_cols(cb, out_ref):
        @pl.loop(0, SLICE // 16, unroll=2)
        def _(k):
            a = cb[0, pl.ds(k * 16, 16)]
            for j in range(1, NS):
                a = a + cb[j, pl.ds(k * 16, 16)]
            out_ref[pl.ds(k * 16, 16)] = a

    def to2d(src_ref, dst2):
        for r in range(SROWS):
            @pl.loop(0, 8)
            def _(c):
                dst2[r, pl.ds(c * 16, 16)] = src_ref[pl.ds(r * 128 + c * 16,
                                                           16)]

    zero(acc, 2 * NPAD)

    @pl.loop(0, EPT // 16, unroll=4)
    def _(i):
        d16 = dst_v[pl.ds(i * 16, 16)]
        plsc.addupdate_scatter(acc, [d16], ones16)

    pltpu.sync_copy(acc.at[pl.ds(0, NPAD)], hacc0.at[tid])
    plsc.subcore_barrier()
    pltpu.sync_copy(hacc0.at[:, pl.ds(base_n, SLICE)], colbuf)
    reduce_cols(colbuf, tmp_v)

    @pl.loop(0, SLICE // 16, unroll=2)
    def _(k):
        deg = tmp_v[pl.ds(k * 16, 16)] + 1.0
        i32 = plsc.bitcast(deg, jnp.int32)
        i32 = jnp.int32(0x5F3759DF) - lax.shift_right_logical(i32, 1)
        y = plsc.bitcast(i32, jnp.float32)
        half = deg * 0.5
        for _ in range(3):
            y = y * (1.5 - half * y * y)
        tdinv[pl.ds(k * 16, 16)] = y
        twx[pl.ds(k * 16, 16)] = y * x_sv[pl.ds(k * 16, 16)]

    pltpu.sync_copy(tdinv, hb2.at[pl.ds(base_n, SLICE)])
    pltpu.sync_copy(twx, hb2.at[pl.ds(NPAD + base_n, SLICE)])
    plsc.subcore_barrier()
    pltpu.sync_copy(hb2, tab)

    zero(acc, 2 * NPAD)
    npad16 = jnp.full((16,), NPAD, jnp.int32)

    @pl.loop(0, EPT // 16, unroll=4)
    def _(i):
        s16 = src_v[pl.ds(i * 16, 16)]
        d16 = dst_v[pl.ds(i * 16, 16)]
        g_d = plsc.load_gather(tab, [s16])
        g_wx = plsc.load_gather(tab, [s16 + npad16])
        plsc.addupdate_scatter(acc, [d16], g_wx)
        plsc.addupdate_scatter(acc, [d16 + npad16], g_d)

    pltpu.sync_copy(acc, hacc2.at[tid])
    plsc.subcore_barrier()
    pltpu.sync_copy(hacc2.at[:, pl.ds(base_n, SLICE)], colbuf)
    reduce_cols(colbuf, tmp_v)

    @pl.loop(0, SLICE // 16, unroll=2)
    def _(k):
        dv = tdinv[pl.ds(k * 16, 16)]
        s1 = dv * (tmp_v[pl.ds(k * 16, 16)] + twx[pl.ds(k * 16, 16)])
        tz[pl.ds(k * 16, 16)] = dv * s1

    pltpu.sync_copy(tz, hbz.at[pl.ds(base_n, SLICE)])
    plsc.subcore_barrier()
    pltpu.sync_copy(hbz, tab_z)

    pltpu.sync_copy(hacc2.at[:, pl.ds(NPAD + base_n, SLICE)], colbuf)
    reduce_cols(colbuf, tmp_v)

    @pl.loop(0, SLICE // 16, unroll=2)
    def _(k):
        dv = tdinv[pl.ds(k * 16, 16)]
        tmp_v[pl.ds(k * 16, 16)] = dv * (tmp_v[pl.ds(k * 16, 16)] + dv)

    to2d(tmp_v, tmp2)
    pltpu.sync_copy(tmp2, t_h.at[pl.ds(tid * SROWS, SROWS), :])

    zero(acc, NPAD)

    @pl.loop(0, EPT // 16, unroll=4)
    def _(i):
        s16 = src_v[pl.ds(i * 16, 16)]
        d16 = dst_v[pl.ds(i * 16, 16)]
        g_z = plsc.load_gather(tab_z, [s16])
        plsc.addupdate_scatter(acc, [d16], g_z)

    pltpu.sync_copy(acc.at[pl.ds(0, NPAD)], hacc0.at[tid])
    plsc.subcore_barrier()
    pltpu.sync_copy(hacc0.at[:, pl.ds(base_n, SLICE)], colbuf)
    reduce_cols(colbuf, tmp_v)

    @pl.loop(0, SLICE // 16, unroll=2)
    def _(k):
        dv = tdinv[pl.ds(k * 16, 16)]
        tmp_v[pl.ds(k * 16, 16)] = dv * (tmp_v[pl.ds(k * 16, 16)]
                                         + tz[pl.ds(k * 16, 16)])

    to2d(tmp_v, tmp2)
    pltpu.sync_copy(tmp2, s2_h.at[pl.ds(tid * SROWS, SROWS), :])


_sc_graph = pl.kernel(
    _sc_body,
    out_type=(
        jax.ShapeDtypeStruct((NPAD // 128, 128), jnp.float32),
        jax.ShapeDtypeStruct((NPAD // 128, 128), jnp.float32),
        jax.ShapeDtypeStruct((NS, NPAD), jnp.float32),
        jax.ShapeDtypeStruct((NS, 2 * NPAD), jnp.float32),
        jax.ShapeDtypeStruct((2 * NPAD,), jnp.float32),
        jax.ShapeDtypeStruct((NPAD,), jnp.float32),
    ),
    mesh=plsc.VectorSubcoreMesh(
        core_axis_name="c", subcore_axis_name="s", num_cores=1,
        num_subcores=NS),
    compiler_params=pltpu.CompilerParams(needs_layout_passes=False),
    scratch_types=[
        pltpu.VMEM((EPT,), jnp.int32),
        pltpu.VMEM((EPT,), jnp.int32),
        pltpu.VMEM((2 * NPAD,), jnp.float32),
        pltpu.VMEM((NPAD,), jnp.float32),
        pltpu.VMEM((SLICE,), jnp.float32),
        pltpu.VMEM((2 * NPAD,), jnp.float32),
        pltpu.VMEM((NS, SLICE), jnp.float32),
        pltpu.VMEM((SLICE,), jnp.float32),
        pltpu.VMEM((SLICE,), jnp.float32),
        pltpu.VMEM((SLICE,), jnp.float32),
        pltpu.VMEM((SLICE,), jnp.float32),
        pltpu.VMEM((SROWS, 128), jnp.float32),
    ],
)


def _uc_body(w1_ref, b1_ref, W2_ref, u_ref, c_ref):
    u_ref[...] = jnp.dot(w1_ref[...], W2_ref[...],
                         preferred_element_type=jnp.float32)
    c_ref[...] = jnp.dot(b1_ref[...], W2_ref[...],
                         preferred_element_type=jnp.float32)


_uc = pl.pallas_call(
    _uc_body,
    out_shape=(
        jax.ShapeDtypeStruct((1, H2), jnp.float32),
        jax.ShapeDtypeStruct((1, H2), jnp.float32),
    ),
)


ROWS_BLK = 1024
RB8 = ROWS_BLK // 128


def _main_body(s2f_ref, tf_ref, u_ref, c_ref, gamma_ref, beta_ref, l1w_ref,
               l1b_ref, l2w_ref, l2b_ref, s2_ref, t_ref, o_ref,
               p_s, q_s, r_s):
    i = pl.program_id(0)

    @pl.when(i == 0)
    def _():
        rows = lax.broadcasted_iota(jnp.int32, (NPAD // 128, 128), 0)
        cols = lax.broadcasted_iota(jnp.int32, (NPAD // 128, 128), 1)
        mask = (rows * 128 + cols) < N

        s2 = jnp.where(mask, s2f_ref[...], 0.0)
        t = jnp.where(mask, tf_ref[...], 0.0)
        inv_n = 1.0 / N
        m_s = jnp.sum(s2) * inv_n
        m_t = jnp.sum(t) * inv_n
        ds = jnp.where(mask, s2 - m_s, 0.0)
        dt = jnp.where(mask, t - m_t, 0.0)
        vs = jnp.sum(ds * ds) * inv_n
        vt = jnp.sum(dt * dt) * inv_n
        cv = jnp.sum(ds * dt) * inv_n

        u = u_ref[...]
        c = c_ref[...]
        var = vs * u * u + vt * c * c + 2.0 * cv * u * c
        scale = gamma_ref[...] / jnp.sqrt(var + EPS)

        p = jnp.dot(u * scale, l1w_ref[...],
                    preferred_element_type=jnp.float32)
        q = jnp.dot(c * scale, l1w_ref[...],
                    preferred_element_type=jnp.float32)
        r = jnp.dot(beta_ref[...], l1w_ref[...],
                    preferred_element_type=jnp.float32) + l1b_ref[...]
        p_s[...] = p
        q_s[...] = q
        r_s[...] = r - m_s * p - m_t * q

    na = lax.broadcasted_iota(jnp.int32, (ROWS_BLK, RB8), 0)
    ka = lax.broadcasted_iota(jnp.int32, (ROWS_BLK, RB8), 1)
    A = jnp.where(lax.shift_right_logical(na, 7) == ka, 1.0, 0.0)
    nd = lax.broadcasted_iota(jnp.int32, (ROWS_BLK, 128), 0)
    cd = lax.broadcasted_iota(jnp.int32, (ROWS_BLK, 128), 1)
    Dm = jnp.where((nd & 127) == cd, 1.0, 0.0)
    gs = jnp.dot(A, s2_ref[...], preferred_element_type=jnp.float32)
    gt = jnp.dot(A, t_ref[...], preferred_element_type=jnp.float32)
    s2c = jnp.sum(gs * Dm, axis=1, keepdims=True)
    tc = jnp.sum(gt * Dm, axis=1, keepdims=True)
    h = s2c * p_s[...] + tc * q_s[...] + r_s[...]
    h = jnp.where(h > 0, h, NEG_SLOPE * h)
    logits = jnp.dot(h, l2w_ref[...],
                     preferred_element_type=jnp.float32) + l2b_ref[...]
    m = jnp.max(logits, axis=1, keepdims=True)
    z = logits - m
    lse = jnp.log(jnp.sum(jnp.exp(z), axis=1, keepdims=True))
    o_ref[...] = z - lse


_main = pl.pallas_call(
    _main_body,
    grid=((N + ROWS_BLK - 1) // ROWS_BLK,),
    in_specs=[
        pl.BlockSpec((NPAD // 128, 128), lambda i: (0, 0)),
        pl.BlockSpec((NPAD // 128, 128), lambda i: (0, 0)),
        pl.BlockSpec((1, H2), lambda i: (0, 0)),
        pl.BlockSpec((1, H2), lambda i: (0, 0)),
        pl.BlockSpec((1, H2), lambda i: (0, 0)),
        pl.BlockSpec((1, H2), lambda i: (0, 0)),
        pl.BlockSpec((H2, H3), lambda i: (0, 0)),
        pl.BlockSpec((1, H3), lambda i: (0, 0)),
        pl.BlockSpec((H3, OUT), lambda i: (0, 0)),
        pl.BlockSpec((1, OUT), lambda i: (0, 0)),
        pl.BlockSpec((RB8, 128), lambda i: (i, 0)),
        pl.BlockSpec((RB8, 128), lambda i: (i, 0)),
    ],
    out_specs=pl.BlockSpec((ROWS_BLK, OUT), lambda i: (i, 0)),
    out_shape=jax.ShapeDtypeStruct((N, OUT), jnp.float32),
    scratch_shapes=[
        pltpu.VMEM((1, H3), jnp.float32),
        pltpu.VMEM((1, H3), jnp.float32),
        pltpu.VMEM((1, H3), jnp.float32),
    ],
)


def kernel(x, edge_index, W1, b1, W2, b2, gamma, beta, lin1_W, lin1_b,
           lin2_W, lin2_b):
    del b2
    xf = x.reshape(N).astype(jnp.float32)

    u, c = _uc(W1.reshape(1, H1), b1.reshape(1, H1), W2)
    s2p, tp, _, _, _, _ = _sc_graph(edge_index.reshape(2 * E), xf)

    return _main(
        s2p,
        tp,
        u,
        c,
        gamma.reshape(1, H2),
        beta.reshape(1, H2),
        lin1_W,
        lin1_b.reshape(1, H3),
        lin2_W,
        lin2_b.reshape(1, OUT),
        s2p,
        tp,
    )
